# Initial kernel scaffold; baseline (speedup 1.0000x reference)
import jax
import jax.numpy as jnp
from jax.experimental import pallas as pl


def kernel(seq, embed_table, W1, b1, W2, b2, gamma, beta, Wg, bg, Wq, bq, Wo, bo):
    raise NotImplementedError("write your pallas kernel here")


# speedup vs baseline: 99.0463x; 99.0463x over previous
import functools

import jax
import jax.numpy as jnp
from jax import lax
from jax.experimental import pallas as pl
from jax.experimental.pallas import tpu as pltpu
from jax.experimental.pallas import tpu_sc as plsc

_NC, _NS, _LANES = 2, 16, 16
_NW = _NC * _NS


def _make_hist(B, L, V):
    rows_per_w = B // _NW
    words_per_w = rows_per_w * L
    steps = L // _LANES
    mesh = plsc.VectorSubcoreMesh(
        core_axis_name="c", subcore_axis_name="s",
        num_cores=_NC, num_subcores=_NS)

    @functools.partial(
        pl.kernel,
        out_type=jax.ShapeDtypeStruct((B * V,), jnp.float32),
        mesh=mesh,
        scratch_types=[
            pltpu.VMEM((words_per_w,), jnp.int32),
            pltpu.VMEM((rows_per_w * V * _LANES,), jnp.float32),
            pltpu.VMEM((rows_per_w * V,), jnp.float32),
        ],
        compiler_params=pltpu.CompilerParams(needs_layout_passes=False),
    )
    def hist(seq_hbm, out_hbm, seq_v, hist_v, cnt_v):
        wid = lax.axis_index("s") * _NC + lax.axis_index("c")
        pltpu.sync_copy(seq_hbm.at[pl.ds(wid * words_per_w, words_per_w)], seq_v)
        zeros = jnp.zeros((_LANES,), jnp.float32)
        for j in range(rows_per_w * V):
            hist_v[pl.ds(j * _LANES, _LANES)] = zeros
        lane = lax.iota(jnp.int32, _LANES)
        ones = jnp.ones((_LANES,), jnp.float32)
        for r in range(rows_per_w):
            roff = r * V * _LANES
            soff = r * L

            def body(i, carry, soff=soff, roff=roff):
                vals = seq_v[pl.ds(soff + i * _LANES, _LANES)]
                plsc.addupdate_scatter(
                    hist_v, [vals * _LANES + lane + roff], ones)
                return carry

            lax.fori_loop(0, steps, body, 0, unroll=8)
            for c in range(V // _LANES):
                base = lane * _LANES + (roff + c * _LANES * _LANES)
                acc = plsc.load_gather(hist_v, [base])
                for j in range(1, _LANES):
                    acc = acc + plsc.load_gather(hist_v, [base + j])
                cnt_v[pl.ds(r * V + c * _LANES, _LANES)] = acc
        pltpu.sync_copy(
            cnt_v, out_hbm.at[pl.ds(wid * rows_per_w * V, rows_per_w * V)])

    return hist


def _dense_body(counts_ref, last_ref, e_ref, w1_ref, b1_ref, w2_ref, b2_ref,
                gamma_ref, beta_ref, wg_ref, wq_ref, bq_ref, wo_ref, bo_ref,
                out_ref, *, B, V, K):
    f32 = jnp.float32
    E = e_ref[...]
    ff = jnp.maximum(
        jnp.dot(E, w1_ref[...], preferred_element_type=f32) + b1_ref[...], 0.0)
    T = E + jnp.dot(ff, w2_ref[...], preferred_element_type=f32) + b2_ref[...]
    mu = jnp.mean(T, axis=-1, keepdims=True)
    var = jnp.mean((T - mu) ** 2, axis=-1, keepdims=True)
    Tn = (T - mu) * lax.rsqrt(var + 1e-5) * gamma_ref[...] + beta_ref[...]
    c = jnp.dot(Tn, wg_ref[...], preferred_element_type=f32)
    c_row = lax.dot_general(jnp.ones((V, 1), f32), c,
                            (((1,), (1,)), ((), ())),
                            preferred_element_type=f32)
    P = (jnp.broadcast_to(c, (V, V)) > c_row).astype(f32)
    counts = counts_ref[...]
    N = jnp.dot(counts, P, preferred_element_type=f32)
    m = jnp.minimum(jnp.maximum(K - N, 0.0), counts)
    last = last_ref[...]
    onehot = (jnp.broadcast_to(last, (B, V))
              == lax.broadcasted_iota(jnp.int32, (B, V), 1)).astype(f32)
    qh = jnp.dot(onehot, Tn, preferred_element_type=f32)
    q = jnp.dot(qh, wq_ref[...], preferred_element_type=f32) + bq_ref[...]
    s = lax.dot_general(q, Tn, (((1,), (1,)), ((), ())),
                        preferred_element_type=f32)
    mx = jnp.max(jnp.where(m > 0.0, s, -1e30), axis=-1, keepdims=True)
    w = m * jnp.exp(jnp.minimum(s - mx, 0.0))
    pooled = (jnp.dot(w, Tn, preferred_element_type=f32)
              / jnp.sum(w, axis=-1, keepdims=True))
    out_ref[...] = (jnp.dot(pooled, wo_ref[...], preferred_element_type=f32)
                    + bo_ref[...])


def kernel(seq, embed_table, W1, b1, W2, b2, gamma, beta, Wg, bg, Wq, bq, Wo,
           bo):
    B, L = seq.shape
    V, d = embed_table.shape
    seq = seq.astype(jnp.int32)
    counts = _make_hist(B, L, V)(seq.reshape(B * L)).reshape(B, V)
    last = lax.slice(seq, (0, L - 1), (B, L))
    body = functools.partial(_dense_body, B=B, V=V, K=float(min(8, L)))
    return pl.pallas_call(
        body,
        out_shape=jax.ShapeDtypeStruct((B, Wo.shape[1]), jnp.float32),
    )(counts, last, embed_table, W1, b1.reshape(1, -1), W2, b2.reshape(1, -1),
      gamma.reshape(1, -1), beta.reshape(1, -1), Wg, Wq, bq.reshape(1, -1),
      Wo, bo.reshape(1, -1))


# speedup vs baseline: 162.3760x; 1.6394x over previous
import functools

import jax
import jax.numpy as jnp
from jax import lax
from jax.experimental import pallas as pl
from jax.experimental.pallas import tpu as pltpu
from jax.experimental.pallas import tpu_sc as plsc

_NC, _NS, _LANES = 2, 16, 16
_NW = _NC * _NS


def _make_hist(B, L, V):
    rows_per_w = B // _NW
    steps = L // _LANES
    mesh = plsc.VectorSubcoreMesh(
        core_axis_name="c", subcore_axis_name="s",
        num_cores=_NC, num_subcores=_NS)

    @functools.partial(
        pl.kernel,
        out_type=jax.ShapeDtypeStruct((B, V), jnp.float32),
        mesh=mesh,
        scratch_types=[
            pltpu.VMEM((rows_per_w, L), jnp.int32),
            pltpu.VMEM((rows_per_w * V * _LANES,), jnp.float32),
            pltpu.VMEM((rows_per_w, V), jnp.float32),
            pltpu.SemaphoreType.DMA,
            pltpu.SemaphoreType.DMA,
        ],
        compiler_params=pltpu.CompilerParams(needs_layout_passes=False),
    )
    def hist(seq_hbm, out_hbm, seq_v, hist_v, cnt_v, sem0, sem1):
        wid = lax.axis_index("s") * _NC + lax.axis_index("c")
        row0 = wid * rows_per_w
        half = rows_per_w // 2
        cp0 = pltpu.async_copy(seq_hbm.at[pl.ds(row0, half)],
                               seq_v.at[pl.ds(0, half)], sem0)
        cp1 = pltpu.async_copy(seq_hbm.at[pl.ds(row0 + half, half)],
                               seq_v.at[pl.ds(half, half)], sem1)
        zeros = jnp.zeros((_LANES,), jnp.float32)
        for j in range(rows_per_w * V):
            hist_v[pl.ds(j * _LANES, _LANES)] = zeros
        lane = lax.iota(jnp.int32, _LANES)
        ones = jnp.ones((_LANES,), jnp.float32)
        lane16 = lane * _LANES
        nchunks = V // _LANES
        ch = 16
        n = steps // ch

        def do_row(r):
            roff = r * (V * _LANES)
            laneoff = lane + roff

            def load(i, j):
                return seq_v[r, pl.ds(i * (_LANES * ch) + j * _LANES,
                                      _LANES)]

            def scat(v):
                plsc.addupdate_scatter(hist_v, [v * _LANES + laneoff], ones)

            def body(i, vals):
                nxt = tuple(load(i + 1, j) for j in range(ch))
                for v in vals:
                    scat(v)
                return nxt

            last = lax.fori_loop(0, n - 1, body,
                                 tuple(load(0, j) for j in range(ch)))
            for v in last:
                scat(v)
            bases = [lane16 + (roff + c * _LANES * _LANES)
                     for c in range(nchunks)]

            def ebody(j, accs, bases=bases):
                return tuple(a + plsc.load_gather(hist_v, [b + j])
                             for a, b in zip(accs, bases))

            accs = lax.fori_loop(0, _LANES, ebody, (zeros,) * nchunks,
                                 unroll=2)
            for c in range(nchunks):
                cnt_v[r, pl.ds(c * _LANES, _LANES)] = accs[c]

        cp0.wait()
        lax.fori_loop(0, half, lambda r, c: (do_row(r), c)[1], 0)
        cp1.wait()
        lax.fori_loop(half, rows_per_w, lambda r, c: (do_row(r), c)[1], 0)
        pltpu.sync_copy(cnt_v,
                        out_hbm.at[pl.ds(wid * rows_per_w, rows_per_w)])

    return hist


def _table_body(e_ref, w1_ref, b1_ref, w2_ref, b2_ref, gamma_ref, beta_ref,
                wg_ref, wq_ref, tn_ref, tnwq_ref, c_ref):
    f32 = jnp.float32
    E = e_ref[...]
    ff = jnp.maximum(
        jnp.dot(E, w1_ref[...], preferred_element_type=f32) + b1_ref[...], 0.0)
    T = E + jnp.dot(ff, w2_ref[...], preferred_element_type=f32) + b2_ref[...]
    mu = jnp.mean(T, axis=-1, keepdims=True)
    var = jnp.mean((T - mu) ** 2, axis=-1, keepdims=True)
    Tn = (T - mu) * lax.rsqrt(var + 1e-5) * gamma_ref[...] + beta_ref[...]
    tn_ref[...] = Tn
    tnwq_ref[...] = jnp.dot(Tn, wq_ref[...], preferred_element_type=f32)
    c_ref[...] = jnp.dot(Tn, wg_ref[...], preferred_element_type=f32)


def _final_body(counts_ref, seqtail_ref, tn_ref, tnwq_ref, c_ref, bq_ref,
                wo_ref, bo_ref, out_ref, *, B, V, K):
    f32 = jnp.float32
    Tn = tn_ref[...]
    c = c_ref[...]
    c_row = lax.dot_general(jnp.ones((V, 1), f32), c,
                            (((1,), (1,)), ((), ())),
                            preferred_element_type=f32)
    P = (jnp.broadcast_to(c, (V, V)) > c_row).astype(f32)
    counts = counts_ref[...]
    N = jnp.dot(counts, P, preferred_element_type=f32)
    m = jnp.minimum(jnp.maximum(K - N, 0.0), counts)
    tail = seqtail_ref[...]
    last = tail[:, -1:]
    onehot = (jnp.broadcast_to(last, (B, V))
              == lax.broadcasted_iota(jnp.int32, (B, V), 1)).astype(f32)
    q = (jnp.dot(onehot, tnwq_ref[...], preferred_element_type=f32)
         + bq_ref[...])
    s = lax.dot_general(q, Tn, (((1,), (1,)), ((), ())),
                        preferred_element_type=f32)
    mx = jnp.max(jnp.where(m > 0.0, s, -1e30), axis=-1, keepdims=True)
    w = m * jnp.exp(jnp.minimum(s - mx, 0.0))
    pooled = (jnp.dot(w, Tn, preferred_element_type=f32)
              / jnp.sum(w, axis=-1, keepdims=True))
    out_ref[...] = (lax.dot_general(wo_ref[...], pooled,
                                    (((0,), (1,)), ((), ())),
                                    preferred_element_type=f32)
                    + bo_ref[...])


def kernel(seq, embed_table, W1, b1, W2, b2, gamma, beta, Wg, bg, Wq, bq, Wo,
           bo):
    B, L = seq.shape
    V, d = embed_table.shape
    seq = seq.astype(jnp.int32)
    counts = _make_hist(B, L, V)(seq)
    full = lambda a: pl.BlockSpec(a.shape, lambda i: (0,) * a.ndim)
    targs = (embed_table, W1, b1.reshape(1, -1), W2, b2.reshape(1, -1),
             gamma.reshape(1, -1), beta.reshape(1, -1), Wg, Wq)
    f32 = jnp.float32
    Tn, TnWq, c = pl.pallas_call(
        _table_body,
        grid=(1,),
        in_specs=[full(a) for a in targs],
        out_specs=[pl.BlockSpec((V, d), lambda i: (0, 0)),
                   pl.BlockSpec((V, d), lambda i: (0, 0)),
                   pl.BlockSpec((V, 1), lambda i: (0, 0))],
        out_shape=[jax.ShapeDtypeStruct((V, d), f32),
                   jax.ShapeDtypeStruct((V, d), f32),
                   jax.ShapeDtypeStruct((V, 1), f32)],
    )(*targs)
    body = functools.partial(_final_body, B=B, V=V, K=float(min(8, L)))
    args = (counts, seq, Tn, TnWq, c, bq.reshape(1, -1), Wo,
            bo.reshape(-1, 1))
    in_specs = [full(a) for a in args]
    in_specs[1] = pl.BlockSpec((B, 128), lambda i: (0, L // 128 - 1))
    out_t = pl.pallas_call(
        body,
        grid=(1,),
        in_specs=in_specs,
        out_specs=pl.BlockSpec((Wo.shape[1], B), lambda i: (0, 0)),
        out_shape=jax.ShapeDtypeStruct((Wo.shape[1], B), jnp.float32),
    )(*args)
    return jnp.transpose(out_t)
